# Initial kernel scaffold; baseline (speedup 1.0000x reference)
import jax
import jax.numpy as jnp
from jax.experimental import pallas as pl


def kernel(token_indices, table):
    raise NotImplementedError("write your pallas kernel here")


# speedup vs baseline: 1.2148x; 1.2148x over previous
import functools

import jax
import jax.numpy as jnp
from jax import lax
from jax.experimental import pallas as pl
from jax.experimental.pallas import tpu as pltpu
from jax.experimental.pallas import tpu_sc as plsc

VOCAB = 1000000
DIM = 32
BATCH = 4096
HIST = 200
B = BATCH * HIST

NC = 2
NS = 16
NW = NC * NS
B_PER_W = B // NW

CHUNK = 128
NCHUNK = B_PER_W // CHUNK
UNROLL = 8

_mesh = plsc.VectorSubcoreMesh(core_axis_name="c", subcore_axis_name="s")


@functools.partial(
    pl.kernel,
    mesh=_mesh,
    out_type=jax.ShapeDtypeStruct((B, DIM), jnp.float32),
    scratch_types=[
        pltpu.VMEM((UNROLL, CHUNK), jnp.int32),
        pltpu.VMEM((UNROLL, CHUNK, DIM), jnp.float32),
        pltpu.SemaphoreType.DMA,
    ],
    compiler_params=pltpu.CompilerParams(use_tc_tiling_on_sc=False),
)
def _emb_gather(idx_hbm, table_hbm, out_hbm, idx_v, rows_v, sem):
    wid = lax.axis_index("s") * NC + lax.axis_index("c")
    base = wid * B_PER_W

    @pl.loop(0, NCHUNK, step=UNROLL)
    def _chunks(c0):
        for u in range(UNROLL):
            off = base + (c0 + u) * CHUNK
            pltpu.sync_copy(idx_hbm.at[pl.ds(off, CHUNK)], idx_v.at[u])
            pltpu.async_copy(table_hbm.at[idx_v.at[u]], rows_v.at[u], sem).wait()
            pltpu.sync_copy(rows_v.at[u], out_hbm.at[pl.ds(off, CHUNK)])


def kernel(token_indices, table):
    idx = token_indices.reshape(B).astype(jnp.int32)
    out = _emb_gather(idx, table)
    return out.reshape(BATCH, HIST, DIM)


# speedup vs baseline: 1.5005x; 1.2352x over previous
import functools

import jax
import jax.numpy as jnp
from jax import lax
from jax.experimental import pallas as pl
from jax.experimental.pallas import tpu as pltpu
from jax.experimental.pallas import tpu_sc as plsc

VOCAB = 1000000
DIM = 32
BATCH = 4096
HIST = 200
B = BATCH * HIST

NC = 2
NS = 16
NW = NC * NS
B_PER_W = B // NW

CHUNK = 128
NCHUNK = B_PER_W // CHUNK
K = 10
HALF = K * CHUNK
NGROUP = NCHUNK // K

_mesh = plsc.VectorSubcoreMesh(core_axis_name="c", subcore_axis_name="s")


@functools.partial(
    pl.kernel,
    mesh=_mesh,
    out_type=jax.ShapeDtypeStruct((B, DIM), jnp.float32),
    scratch_types=[
        pltpu.VMEM((NCHUNK, CHUNK), jnp.int32),
        pltpu.VMEM((2, HALF, DIM), jnp.float32),
        pltpu.SemaphoreType.DMA,
        pltpu.SemaphoreType.DMA,
        pltpu.SemaphoreType.DMA,
        pltpu.SemaphoreType.DMA,
    ],
    compiler_params=pltpu.CompilerParams(use_tc_tiling_on_sc=False),
)
def _emb_gather(idx_hbm, table_hbm, out_hbm, idx_v, rows_v, gs0, gs1, os0, os1):
    wid = lax.axis_index("s") * NC + lax.axis_index("c")
    base = wid * B_PER_W
    gsem = (gs0, gs1)
    osem = (os0, os1)

    pltpu.sync_copy(idx_hbm.at[wid], idx_v)

    def fire_gathers(g, h):
        for j in range(K):
            pltpu.async_copy(
                table_hbm.at[idx_v.at[g * K + j]],
                rows_v.at[h].at[pl.ds(j * CHUNK, CHUNK)],
                gsem[h],
            )

    def process(g, h, refill):
        out_slice = out_hbm.at[pl.ds(base + g * HALF, HALF)]
        pltpu.make_async_copy(out_slice, rows_v.at[h], gsem[h]).wait()
        out_copy = pltpu.make_async_copy(rows_v.at[h], out_slice, osem[h])
        out_copy.start()
        out_copy.wait()
        if refill:
            fire_gathers(g + 2, h)

    fire_gathers(0, 0)
    fire_gathers(1, 1)

    @pl.loop(0, NGROUP - 2, step=2)
    def _groups(g0):
        process(g0, 0, True)
        process(g0 + 1, 1, True)

    process(NGROUP - 2, 0, False)
    process(NGROUP - 1, 1, False)


def kernel(token_indices, table):
    idx = token_indices.reshape(NW, NCHUNK, CHUNK).astype(jnp.int32)
    out = _emb_gather(idx, table)
    return out.reshape(BATCH, HIST, DIM)
